# Initial kernel scaffold; baseline (speedup 1.0000x reference)
import functools

import jax
import jax.numpy as jnp
from jax import lax
from jax.experimental import pallas as pl
from jax.experimental.pallas import tpu as pltpu
from jax.experimental.pallas import tpu_sc as plsc

N = 10000
E = 320000
NODE_DIM = 128
EDGE_DIM = 16
GLOBAL_DIM = 16
HID = 64
EPS = 1e-5

NC = 2
NS = 16
NW = NC * NS
EPT = E // NW
KE = 80
RPT = N // NS


def _ln(u, g, b):
    mu = jnp.mean(u, axis=-1, keepdims=True)
    var = jnp.mean((u - mu) ** 2, axis=-1, keepdims=True)
    return (u - mu) / jnp.sqrt(var + EPS) * g + b


def _make_sc_spmm(with_ea):
    mesh = plsc.VectorSubcoreMesh(core_axis_name="c", subcore_axis_name="s")
    out_type = [jax.ShapeDtypeStruct((N, HID), jnp.float32),
                jax.ShapeDtypeStruct((N, HID), jnp.float32)]
    scratch = [
        pltpu.VMEM((KE,), jnp.int32),
        pltpu.VMEM((KE,), jnp.int32),
        pltpu.VMEM((KE, HID), jnp.float32),
        pltpu.VMEM((RPT, HID), jnp.float32),
        pltpu.VMEM_SHARED((N, HID), jnp.float32),
        pltpu.SemaphoreType.DMA,
    ]
    if with_ea:
        out_type += [jax.ShapeDtypeStruct((N, EDGE_DIM), jnp.float32),
                     jax.ShapeDtypeStruct((N, EDGE_DIM), jnp.float32)]
        scratch += [
            pltpu.VMEM((KE, EDGE_DIM), jnp.float32),
            pltpu.VMEM((RPT, EDGE_DIM), jnp.float32),
            pltpu.VMEM_SHARED((N, EDGE_DIM), jnp.float32),
        ]

    def body(*refs):
        if with_ea:
            (t_hbm, src_hbm, dst_hbm, ea_hbm, s0, s1, e0, e1,
             sidx, didx, rows, zbuf, acc, sem, erows, ezbuf, eacc) = refs
        else:
            (t_hbm, src_hbm, dst_hbm, s0, s1,
             sidx, didx, rows, zbuf, acc, sem) = refs
        c = lax.axis_index("c")
        s = lax.axis_index("s")
        wid = s * NC + c
        zero16 = jnp.zeros((16,), jnp.float32)

        def zrow(i, _):
            for j in range(HID // 16):
                zbuf[i, pl.ds(j * 16, 16)] = zero16
            if with_ea:
                ezbuf[i, pl.ds(0, 16)] = zero16
            return 0

        lax.fori_loop(0, RPT, zrow, 0)
        rsl = pl.ds(s * RPT, RPT)
        pltpu.sync_copy(zbuf, acc.at[rsl, :])
        if with_ea:
            pltpu.sync_copy(ezbuf, eacc.at[rsl, :])
        plsc.subcore_barrier()

        ebase = wid * EPT

        def edge_chunk(i, _):
            b = ebase + i * KE
            pltpu.sync_copy(src_hbm.at[pl.ds(b, KE)], sidx)
            pltpu.async_copy(t_hbm.at[sidx], rows, sem).wait()
            pltpu.sync_copy(dst_hbm.at[pl.ds(b, KE)], didx)
            pltpu.sync_copy(rows, acc.at[didx], add=True)
            if with_ea:
                pltpu.sync_copy(ea_hbm.at[pl.ds(b, KE), :], erows)
                pltpu.sync_copy(erows, eacc.at[didx], add=True)
            return 0

        lax.fori_loop(0, EPT // KE, edge_chunk, 0)
        plsc.subcore_barrier()

        @pl.when(c == 0)
        def _():
            pltpu.sync_copy(acc.at[rsl, :], s0.at[rsl, :])
            if with_ea:
                pltpu.sync_copy(eacc.at[rsl, :], e0.at[rsl, :])

        @pl.when(c == 1)
        def _():
            pltpu.sync_copy(acc.at[rsl, :], s1.at[rsl, :])
            if with_ea:
                pltpu.sync_copy(eacc.at[rsl, :], e1.at[rsl, :])

    return pl.kernel(body, out_type=out_type, mesh=mesh,
                     scratch_types=scratch)


_sc_spmm_ea = _make_sc_spmm(True)
_sc_spmm = _make_sc_spmm(False)


def _tc_a_body(x, w1, b1, w2a, b2, o):
    m = jnp.dot(x[...], w1[...], preferred_element_type=jnp.float32) + b1[...]
    o[...] = jnp.dot(m, w2a[...], preferred_element_type=jnp.float32) + b2[...]


def _aggr_update(s0, s1, e0, e1, gf, w2b, wua, wub, bu, g, bln):
    aggr = s0[...] + s1[...] + jnp.dot(
        e0[...] + e1[...], w2b[...], preferred_element_type=jnp.float32)
    u = jnp.dot(aggr, wua[...], preferred_element_type=jnp.float32)
    u = u + jnp.dot(gf[...], wub[...],
                    preferred_element_type=jnp.float32) + bu[...]
    return _ln(u, g[...], bln[...])


def _tc_b_body(s0, s1, e0, e1, gf, w2b, wua, wub, bu, g, bln,
               w1n, b1n, w2an, b2n, o):
    h = _aggr_update(s0, s1, e0, e1, gf, w2b, wua, wub, bu, g, bln)
    m = jnp.dot(h, w1n[...], preferred_element_type=jnp.float32) + b1n[...]
    o[...] = jnp.dot(m, w2an[...],
                     preferred_element_type=jnp.float32) + b2n[...]


def _tc_b3_body(s0, s1, e0, e1, gf, w2b, wua, wub, bu, g, bln,
                wr1, br1, g1, bl1, wr2, br2, g2, bl2, wr3, br3, o):
    h = _aggr_update(s0, s1, e0, e1, gf, w2b, wua, wub, bu, g, bln)
    pooled = jnp.mean(h, axis=0, keepdims=True)
    z = jnp.concatenate([pooled, gf[...]], axis=1)
    r = jnp.dot(z, wr1[...], preferred_element_type=jnp.float32) + br1[...]
    r = jax.nn.relu(_ln(r, g1[...], bl1[...]))
    r = jnp.dot(r, wr2[...], preferred_element_type=jnp.float32) + br2[...]
    r = jax.nn.relu(_ln(r, g2[...], bl2[...]))
    o[...] = jnp.dot(r, wr3[...], preferred_element_type=jnp.float32) + br3[...]


_f32 = jnp.float32
_tc_a = pl.pallas_call(
    _tc_a_body, out_shape=jax.ShapeDtypeStruct((N, HID), _f32))
_tc_b = pl.pallas_call(
    _tc_b_body, out_shape=jax.ShapeDtypeStruct((N, HID), _f32))
_tc_b3 = pl.pallas_call(
    _tc_b3_body, out_shape=jax.ShapeDtypeStruct((1, 1), _f32))


def _row(v):
    return v.reshape(1, -1)


def kernel(x, edge_attr, global_feature, params, edge_index, batch):
    src = edge_index[0]
    dst = edge_index[1]
    gf = global_feature
    lp = params["layers"]

    def pre(i):
        p = lp[i]
        return (p["W1"], _row(p["b1"]), p["W2"][:HID], _row(p["b2"]))

    def upd(i):
        p = lp[i]
        return (p["W2"][HID:], p["Wu"][:HID], p["Wu"][HID:], _row(p["bu"]),
                _row(p["g"]), _row(p["bln"]))

    t = _tc_a(x, *pre(0))
    s0, s1, e0, e1 = _sc_spmm_ea(t, src, dst, edge_attr)
    t = _tc_b(s0, s1, e0, e1, gf, *upd(0), *pre(1))
    s0, s1 = _sc_spmm(t, src, dst)
    t = _tc_b(s0, s1, e0, e1, gf, *upd(1), *pre(2))
    s0, s1 = _sc_spmm(t, src, dst)
    out = _tc_b3(s0, s1, e0, e1, gf, *upd(2),
                 params["Wr1"], _row(params["br1"]), _row(params["g1"]),
                 _row(params["bln1"]),
                 params["Wr2"], _row(params["br2"]), _row(params["g2"]),
                 _row(params["bln2"]),
                 params["Wr3"], _row(params["br3"]))
    return out[:, 0]


# speedup vs baseline: 4.5134x; 4.5134x over previous
import functools

import jax
import jax.numpy as jnp
from jax import lax
from jax.experimental import pallas as pl
from jax.experimental.pallas import tpu as pltpu
from jax.experimental.pallas import tpu_sc as plsc

N = 10000
E = 320000
NODE_DIM = 128
EDGE_DIM = 16
GLOBAL_DIM = 16
HID = 64
EPS = 1e-5

NC = 2
NS = 16
NW = NC * NS
EPT = E // NW
KE = 80
NP = 10240
RPT = NP // NS


def _ln(u, g, b):
    mu = jnp.mean(u, axis=-1, keepdims=True)
    var = jnp.mean((u - mu) ** 2, axis=-1, keepdims=True)
    return (u - mu) / jnp.sqrt(var + EPS) * g + b


def _make_sc_spmm(with_ea):
    mesh = plsc.VectorSubcoreMesh(core_axis_name="c", subcore_axis_name="s")
    out_type = [jax.ShapeDtypeStruct((NP, HID), jnp.float32),
                jax.ShapeDtypeStruct((NP, HID), jnp.float32)]
    scratch = [
        pltpu.VMEM((KE,), jnp.int32),
        pltpu.VMEM((KE,), jnp.int32),
        pltpu.VMEM((KE, HID), jnp.float32),
        pltpu.VMEM((RPT, HID), jnp.float32),
        pltpu.VMEM_SHARED((NP, HID), jnp.float32),
        pltpu.SemaphoreType.DMA,
    ]
    if with_ea:
        out_type += [jax.ShapeDtypeStruct((NP, EDGE_DIM), jnp.float32),
                     jax.ShapeDtypeStruct((NP, EDGE_DIM), jnp.float32)]
        scratch += [
            pltpu.VMEM((KE, EDGE_DIM), jnp.float32),
            pltpu.VMEM((RPT, EDGE_DIM), jnp.float32),
            pltpu.VMEM_SHARED((NP, EDGE_DIM), jnp.float32),
        ]

    def body(*refs):
        if with_ea:
            (t_hbm, src_hbm, dst_hbm, ea_hbm, s0, s1, e0, e1,
             sidx, didx, rows, zbuf, acc, sem, erows, ezbuf, eacc) = refs
        else:
            (t_hbm, src_hbm, dst_hbm, s0, s1,
             sidx, didx, rows, zbuf, acc, sem) = refs
        c = lax.axis_index("c")
        s = lax.axis_index("s")
        wid = s * NC + c
        zero16 = jnp.zeros((16,), jnp.float32)

        def zrow(i, _):
            for j in range(HID // 16):
                zbuf[i, pl.ds(j * 16, 16)] = zero16
            if with_ea:
                ezbuf[i, pl.ds(0, 16)] = zero16
            return 0

        lax.fori_loop(0, RPT, zrow, 0)
        rsl = pl.ds(s * RPT, RPT)
        pltpu.sync_copy(zbuf, acc.at[rsl, :])
        if with_ea:
            pltpu.sync_copy(ezbuf, eacc.at[rsl, :])
        plsc.subcore_barrier()

        ebase = wid * EPT

        def edge_chunk(i, _):
            b = ebase + i * KE
            pltpu.sync_copy(src_hbm.at[pl.ds(b, KE)], sidx)
            pltpu.async_copy(t_hbm.at[sidx], rows, sem).wait()
            pltpu.sync_copy(dst_hbm.at[pl.ds(b, KE)], didx)
            pltpu.sync_copy(rows, acc.at[didx], add=True)
            if with_ea:
                pltpu.sync_copy(ea_hbm.at[pl.ds(b, KE), :], erows)
                pltpu.sync_copy(erows, eacc.at[didx], add=True)
            return 0

        lax.fori_loop(0, EPT // KE, edge_chunk, 0)
        plsc.subcore_barrier()

        @pl.when(c == 0)
        def _():
            pltpu.sync_copy(acc.at[rsl, :], s0.at[rsl, :])
            if with_ea:
                pltpu.sync_copy(eacc.at[rsl, :], e0.at[rsl, :])

        @pl.when(c == 1)
        def _():
            pltpu.sync_copy(acc.at[rsl, :], s1.at[rsl, :])
            if with_ea:
                pltpu.sync_copy(eacc.at[rsl, :], e1.at[rsl, :])

    return pl.kernel(
        body, out_type=out_type, mesh=mesh, scratch_types=scratch,
        compiler_params=pltpu.CompilerParams(use_tc_tiling_on_sc=False))


@functools.lru_cache(maxsize=None)
def _get_sc_spmm(with_ea):
    return _make_sc_spmm(with_ea)


def _sc_spmm_ea(t, src, dst, ea):
    return _get_sc_spmm(True)(t, src, dst, ea)


def _sc_spmm(t, src, dst):
    return _get_sc_spmm(False)(t, src, dst)


def _tc_a_body(x, w1, b1, w2a, b2, o):
    m = jnp.dot(x[...], w1[...], preferred_element_type=jnp.float32) + b1[...]
    t = jnp.dot(m, w2a[...], preferred_element_type=jnp.float32) + b2[...]
    o[...] = jnp.concatenate(
        [t, jnp.zeros((NP - N, HID), jnp.float32)], axis=0)


def _aggr_update(s0, s1, e0, e1, gf, w2b, wua, wub, bu, g, bln):
    aggr = s0[...] + s1[...] + jnp.dot(
        e0[...] + e1[...], w2b[...], preferred_element_type=jnp.float32)
    u = jnp.dot(aggr, wua[...], preferred_element_type=jnp.float32)
    u = u + jnp.dot(gf[...], wub[...],
                    preferred_element_type=jnp.float32) + bu[...]
    return _ln(u, g[...], bln[...])


def _tc_b_body(s0, s1, e0, e1, gf, w2b, wua, wub, bu, g, bln,
               w1n, b1n, w2an, b2n, o):
    h = _aggr_update(s0, s1, e0, e1, gf, w2b, wua, wub, bu, g, bln)
    m = jnp.dot(h, w1n[...], preferred_element_type=jnp.float32) + b1n[...]
    o[...] = jnp.dot(m, w2an[...],
                     preferred_element_type=jnp.float32) + b2n[...]


def _tc_b3_body(s0, s1, e0, e1, gf, w2b, wua, wub, bu, g, bln,
                wr1, br1, g1, bl1, wr2, br2, g2, bl2, wr3, br3, o):
    h = _aggr_update(s0, s1, e0, e1, gf, w2b, wua, wub, bu, g, bln)
    rows = lax.broadcasted_iota(jnp.int32, (NP, 1), 0)
    pooled = jnp.sum(jnp.where(rows < N, h, 0.0), axis=0,
                     keepdims=True) * (1.0 / N)
    z = jnp.concatenate([pooled, gf[...]], axis=1)
    r = jnp.dot(z, wr1[...], preferred_element_type=jnp.float32) + br1[...]
    r = jax.nn.relu(_ln(r, g1[...], bl1[...]))
    r = jnp.dot(r, wr2[...], preferred_element_type=jnp.float32) + br2[...]
    r = jax.nn.relu(_ln(r, g2[...], bl2[...]))
    o[...] = jnp.dot(r, wr3[...], preferred_element_type=jnp.float32) + br3[...]


_f32 = jnp.float32
_tc_a = pl.pallas_call(
    _tc_a_body, out_shape=jax.ShapeDtypeStruct((NP, HID), _f32))
_tc_b = pl.pallas_call(
    _tc_b_body, out_shape=jax.ShapeDtypeStruct((NP, HID), _f32))
_tc_b3 = pl.pallas_call(
    _tc_b3_body, out_shape=jax.ShapeDtypeStruct((1, 1), _f32))


def _row(v):
    return v.reshape(1, -1)


def kernel(x, edge_attr, global_feature, params, edge_index, batch):
    src = edge_index[0]
    dst = edge_index[1]
    gf = global_feature
    lp = params["layers"]

    def pre(i):
        p = lp[i]
        return (p["W1"], _row(p["b1"]), p["W2"][:HID], _row(p["b2"]))

    def upd(i):
        p = lp[i]
        return (p["W2"][HID:], p["Wu"][:HID], p["Wu"][HID:], _row(p["bu"]),
                _row(p["g"]), _row(p["bln"]))

    t = _tc_a(x, *pre(0))
    s0, s1, e0, e1 = _sc_spmm_ea(t, src, dst, edge_attr)
    t = _tc_b(s0, s1, e0, e1, gf, *upd(0), *pre(1))
    s0, s1 = _sc_spmm(t, src, dst)
    t = _tc_b(s0, s1, e0, e1, gf, *upd(1), *pre(2))
    s0, s1 = _sc_spmm(t, src, dst)
    out = _tc_b3(s0, s1, e0, e1, gf, *upd(2),
                 params["Wr1"], _row(params["br1"]), _row(params["g1"]),
                 _row(params["bln1"]),
                 params["Wr2"], _row(params["br2"]), _row(params["g2"]),
                 _row(params["bln2"]),
                 params["Wr3"], _row(params["br3"]))
    return out[:, 0]
